# baseline (device time: 29156 ns/iter reference)
import jax
import jax.numpy as jnp
from jax import lax
from jax.experimental import pallas as pl
from jax.experimental.pallas import tpu as pltpu

N_DEV = 4
SEND_ORDER = [2, 1, 3]
WAIT_ORDER = [1, 2, 0]


def kernel(x, Win0, Wout0, Win1, Wout1, Win2, Wout2):
    b, d = x.shape
    ck = d // N_DEV
    rows = b // N_DEV

    def body(
        x_ref,
        win0_ref,
        wout0_ref,
        win1_ref,
        wout1_ref,
        win2_ref,
        wout2_ref,
        out_ref,
        rs_out_ref,
        rs_in_ref,
        ag_out_ref,
        ag_in_ref,
        p2_ref,
        rs2_ref,
        rs_send_sems,
        rs_recv_sems,
        ag_send_sems,
        ag_recv_sems,
        rs2_send_sems,
        rs2_recv_sems,
    ):
        my = lax.axis_index("i")

        barrier_sem = pltpu.get_barrier_semaphore()
        for idx in range(1, N_DEV):
            pl.semaphore_signal(
                barrier_sem,
                inc=1,
                device_id=((my + idx) % N_DEV,),
                device_id_type=pl.DeviceIdType.MESH,
            )
        pl.semaphore_wait(barrier_sem, N_DEV - 1)

        wins = [win0_ref, win1_ref, win2_ref]
        wouts = [wout0_ref, wout1_ref, wout2_ref]
        sends = []

        h = jnp.maximum(
            jnp.dot(x_ref[:, :], win0_ref[:, :], preferred_element_type=jnp.float32),
            0.0,
        )

        for l in range(2):
            win_next = wins[l + 1]

            for idx in SEND_ORDER:
                peer = (my + idx) % N_DEV
                pc = jnp.dot(
                    h,
                    wouts[l][:, pl.ds(peer * ck, ck)],
                    preferred_element_type=jnp.float32,
                )
                rs_out_ref[l, idx - 1, :, :] = pc
                rdma = pltpu.make_async_remote_copy(
                    src_ref=rs_out_ref.at[l, idx - 1],
                    dst_ref=rs_in_ref.at[l, N_DEV - 1 - idx],
                    send_sem=rs_send_sems.at[l, idx - 1],
                    recv_sem=rs_recv_sems.at[l, N_DEV - 1 - idx],
                    device_id=(peer,),
                    device_id_type=pl.DeviceIdType.MESH,
                )
                rdma.start()
                sends.append(rdma)
            own = jnp.dot(
                h,
                wouts[l][:, pl.ds(my * ck, ck)],
                preferred_element_type=jnp.float32,
            )
            for j in WAIT_ORDER:
                recv = pltpu.make_async_remote_copy(
                    src_ref=rs_in_ref.at[l, j],
                    dst_ref=rs_in_ref.at[l, j],
                    send_sem=rs_recv_sems.at[l, j],
                    recv_sem=rs_recv_sems.at[l, j],
                    device_id=(my,),
                    device_id_type=pl.DeviceIdType.MESH,
                )
                recv.wait_recv()
                own = own + rs_in_ref[l, j, :, :]

            ag_out_ref[l, :, :] = own
            for idx in SEND_ORDER:
                peer = (my + idx) % N_DEV
                rdma = pltpu.make_async_remote_copy(
                    src_ref=ag_out_ref.at[l],
                    dst_ref=ag_in_ref.at[l, N_DEV - 1 - idx],
                    send_sem=ag_send_sems.at[l, idx - 1],
                    recv_sem=ag_recv_sems.at[l, N_DEV - 1 - idx],
                    device_id=(peer,),
                    device_id_type=pl.DeviceIdType.MESH,
                )
                rdma.start()
                sends.append(rdma)
            acc_h = jnp.dot(
                own,
                win_next[pl.ds(my * ck, ck), :],
                preferred_element_type=jnp.float32,
            )
            for j in WAIT_ORDER:
                recv = pltpu.make_async_remote_copy(
                    src_ref=ag_in_ref.at[l, j],
                    dst_ref=ag_in_ref.at[l, j],
                    send_sem=ag_recv_sems.at[l, j],
                    recv_sem=ag_recv_sems.at[l, j],
                    device_id=(my,),
                    device_id_type=pl.DeviceIdType.MESH,
                )
                recv.wait_recv()
                origin = (my + j + 1) % N_DEV
                acc_h = acc_h + jnp.dot(
                    ag_in_ref[l, j, :, :],
                    win_next[pl.ds(origin * ck, ck), :],
                    preferred_element_type=jnp.float32,
                )
            h = jnp.maximum(acc_h, 0.0)

        p2_ref[:, :] = jnp.dot(
            h, wout2_ref[:, :], preferred_element_type=jnp.float32
        )
        for idx in SEND_ORDER:
            peer = (my + idx) % N_DEV
            rdma = pltpu.make_async_remote_copy(
                src_ref=p2_ref.at[pl.ds(peer * rows, rows)],
                dst_ref=rs2_ref.at[N_DEV - 1 - idx],
                send_sem=rs2_send_sems.at[idx - 1],
                recv_sem=rs2_recv_sems.at[N_DEV - 1 - idx],
                device_id=(peer,),
                device_id_type=pl.DeviceIdType.MESH,
            )
            rdma.start()
            sends.append(rdma)
        own = p2_ref[pl.ds(my * rows, rows), :]
        for j in WAIT_ORDER:
            recv = pltpu.make_async_remote_copy(
                src_ref=rs2_ref.at[j],
                dst_ref=rs2_ref.at[j],
                send_sem=rs2_recv_sems.at[j],
                recv_sem=rs2_recv_sems.at[j],
                device_id=(my,),
                device_id_type=pl.DeviceIdType.MESH,
            )
            recv.wait_recv()
            own = own + rs2_ref[j, :, :]
        out_ref[:, :] = own

        for rdma in sends:
            rdma.wait_send()

    return pl.pallas_call(
        body,
        out_shape=jax.ShapeDtypeStruct((rows, d), jnp.float32),
        in_specs=[pl.BlockSpec(memory_space=pltpu.VMEM)] * 7,
        out_specs=pl.BlockSpec(memory_space=pltpu.VMEM),
        scratch_shapes=[
            pltpu.VMEM((2, N_DEV - 1, b, ck), jnp.float32),
            pltpu.VMEM((2, N_DEV - 1, b, ck), jnp.float32),
            pltpu.VMEM((2, b, ck), jnp.float32),
            pltpu.VMEM((2, N_DEV - 1, b, ck), jnp.float32),
            pltpu.VMEM((b, d), jnp.float32),
            pltpu.VMEM((N_DEV - 1, rows, d), jnp.float32),
            pltpu.SemaphoreType.DMA((2, N_DEV - 1)),
            pltpu.SemaphoreType.DMA((2, N_DEV - 1)),
            pltpu.SemaphoreType.DMA((2, N_DEV - 1)),
            pltpu.SemaphoreType.DMA((2, N_DEV - 1)),
            pltpu.SemaphoreType.DMA((N_DEV - 1,)),
            pltpu.SemaphoreType.DMA((N_DEV - 1,)),
        ],
        compiler_params=pltpu.CompilerParams(collective_id=0),
    )(x, Win0, Wout0, Win1, Wout1, Win2, Wout2)
